# baseline (speedup 1.0000x reference)
import functools

import jax
import jax.numpy as jnp
from jax import lax
from jax.experimental import pallas as pl
from jax.experimental.pallas import tpu as pltpu
from jax.experimental.pallas import tpu_sc as plsc

B, N, D, H, K = 16, 2048, 768, 256, 5
THRESH = 0.5
LANES = 16
CHUNKS = N // LANES
KPAD = 8

_NC = 2
_NS = 16


def _sc_topk_body(pud_hbm, masks_hbm, attn_hbm, emb_hbm,
                  vals_out, idx_out, div_out, attn_out, mask_out, emb_out,
                  pud_v, masks_v, attn_v, masked_v, idx_buf, rows_v,
                  st_vals, st_idx, st_div, st_attn, st_mask, sem):
    c = lax.axis_index("c")
    s = lax.axis_index("s")
    wid = s * _NC + c

    @pl.when(wid < B)
    def _():
        b = wid
        pltpu.sync_copy(pud_hbm.at[b], pud_v)
        pltpu.sync_copy(masks_hbm.at[b], masks_v)
        pltpu.sync_copy(attn_hbm.at[b], attn_v)

        lane = lax.iota(jnp.int32, LANES)

        def _mask_chunk(i, carry):
            v = pud_v[pl.ds(i * LANES, LANES)]
            m = masks_v[pl.ds(i * LANES, LANES)]
            masked_v[pl.ds(i * LANES, LANES)] = jnp.where(m > 0.5, v, 0.0)
            return carry

        lax.fori_loop(0, CHUNKS, _mask_chunk, 0)

        vals_acc = jnp.zeros((LANES,), jnp.float32)
        idx_acc = jnp.zeros((LANES,), jnp.int32)
        for r in range(K):
            def _scan_chunk(i, carry):
                mvec, ivec = carry
                cv = masked_v[pl.ds(i * LANES, LANES)]
                iv = lane + i * LANES
                gt = cv > mvec
                return jnp.where(gt, cv, mvec), jnp.where(gt, iv, ivec)

            mvec, ivec = lax.fori_loop(
                0, CHUNKS, _scan_chunk,
                (jnp.full((LANES,), -1.0, jnp.float32),
                 jnp.zeros((LANES,), jnp.int32)))
            for shift in (8, 4, 2, 1):
                st_vals[...] = mvec
                st_idx[...] = ivec
                perm = jnp.bitwise_xor(lane, shift)
                mv2 = plsc.load_gather(st_vals, [perm])
                iv2 = plsc.load_gather(st_idx, [perm])
                better = (mv2 > mvec) | ((mv2 == mvec) & (iv2 < ivec))
                mvec = jnp.where(better, mv2, mvec)
                ivec = jnp.where(better, iv2, ivec)
            vals_acc = jnp.where(lane == r, mvec, vals_acc)
            idx_acc = jnp.where(lane == r, ivec, idx_acc)
            plsc.store_scatter(masked_v, [ivec],
                               jnp.full((LANES,), -1.0, jnp.float32),
                               mask=lane == 0)

        idx_safe = jnp.where(lane < K, idx_acc, 0)
        st_vals[...] = vals_acc
        st_idx[...] = idx_acc
        st_div[...] = plsc.load_gather(pud_v, [idx_safe])
        st_attn[...] = plsc.load_gather(attn_v, [idx_safe])
        st_mask[...] = plsc.load_gather(masks_v, [idx_safe])

        idx_buf[...] = idx_safe + b * N
        pltpu.async_copy(emb_hbm.at[idx_buf], rows_v, sem).wait()

        pltpu.sync_copy(st_vals, vals_out.at[b])
        pltpu.sync_copy(st_idx, idx_out.at[b])
        pltpu.sync_copy(st_div, div_out.at[b])
        pltpu.sync_copy(st_attn, attn_out.at[b])
        pltpu.sync_copy(st_mask, mask_out.at[b])
        pltpu.sync_copy(rows_v.at[pl.ds(0, KPAD)], emb_out.at[b])


@functools.cache
def _make_sc_topk():
  return pl.kernel(
    _sc_topk_body,
    out_type=[
        jax.ShapeDtypeStruct((B, LANES), jnp.float32),
        jax.ShapeDtypeStruct((B, LANES), jnp.int32),
        jax.ShapeDtypeStruct((B, LANES), jnp.float32),
        jax.ShapeDtypeStruct((B, LANES), jnp.float32),
        jax.ShapeDtypeStruct((B, LANES), jnp.float32),
        jax.ShapeDtypeStruct((B, KPAD, D), jnp.float32),
    ],
    mesh=plsc.VectorSubcoreMesh(core_axis_name="c", subcore_axis_name="s"),
    compiler_params=pltpu.CompilerParams(needs_layout_passes=False),
    scratch_types=[
        pltpu.VMEM((N,), jnp.float32),
        pltpu.VMEM((N,), jnp.float32),
        pltpu.VMEM((N,), jnp.float32),
        pltpu.VMEM((N,), jnp.float32),
        pltpu.VMEM((LANES,), jnp.int32),
        pltpu.VMEM((LANES, D), jnp.float32),
        pltpu.VMEM((LANES,), jnp.float32),
        pltpu.VMEM((LANES,), jnp.int32),
        pltpu.VMEM((LANES,), jnp.float32),
        pltpu.VMEM((LANES,), jnp.float32),
        pltpu.VMEM((LANES,), jnp.float32),
        pltpu.SemaphoreType.DMA,
    ],
  )


def _tc_mlp_body(mean_ref, max_ref, w1_ref, b1_ref, w2_ref, b2_ref, w3_ref,
                 b3_ref, u1a_ref, u1b_ref, ub1_ref, u2d_ref, ub2d_ref,
                 emb_ref, div_ref, vals_ref, mask_ref,
                 score_out, probs_out, flag_out, conf_out):
    f32 = jnp.float32
    meanv = mean_ref[...]
    maxv = max_ref[...]
    w1 = w1_ref[...]
    const_row = 3.0 * w1[2:3, :] + 0.01 * w1[3:4, :] + b1_ref[...]
    h1 = jnp.maximum(meanv * w1[0:1, :] + maxv * w1[1:2, :] + const_row, 0.0)
    h2 = jnp.maximum(
        jnp.dot(h1, w2_ref[...], preferred_element_type=f32) + b2_ref[...],
        0.0)
    logit = jnp.dot(h2, w3_ref[...], preferred_element_type=f32) + b3_ref[...]
    score_out[...] = 1.0 / (1.0 + jnp.exp(-logit))

    emb = emb_ref[...]
    divc = div_ref[...]
    uh = jnp.maximum(
        jnp.dot(emb, u1a_ref[...], preferred_element_type=f32)
        + divc * u1b_ref[...] + ub1_ref[...], 0.0)
    ldiff = jnp.dot(uh, u2d_ref[...], preferred_element_type=f32) + ub2d_ref[...]
    probs = 1.0 / (1.0 + jnp.exp(-ldiff))
    probs_out[...] = probs

    rows = lax.broadcasted_iota(jnp.int32, (B * KPAD, 1), 0)
    valid = (rows % KPAD) < K
    flagf = jnp.where(
        valid & (mask_ref[...] > 0.5) & (vals_ref[...] > THRESH), 1.0, 0.0)
    flag_out[...] = flagf

    cols = lax.broadcasted_iota(jnp.int32, (B, B * KPAD), 1)
    brow = lax.broadcasted_iota(jnp.int32, (B, B * KPAD), 0)
    sel = jnp.where((cols // KPAD) == brow, 1.0, 0.0)
    cnt = jnp.dot(sel, flagf, preferred_element_type=f32)
    sumpf = jnp.dot(sel, probs * flagf, preferred_element_type=f32)
    avg_conf = jnp.where(cnt > 0, sumpf / jnp.maximum(cnt, 1.0), 0.5)
    sep = (maxv - meanv) / (maxv + 1e-8)
    conf = 0.4 * sep + 0.3 * 0.5 + 0.3 * avg_conf
    conf_out[...] = jnp.clip(conf, 0.0, 1.0)


_tc_mlp = pl.pallas_call(
    _tc_mlp_body,
    out_shape=[
        jax.ShapeDtypeStruct((B, 1), jnp.float32),
        jax.ShapeDtypeStruct((B * KPAD, 1), jnp.float32),
        jax.ShapeDtypeStruct((B * KPAD, 1), jnp.float32),
        jax.ShapeDtypeStruct((B, 1), jnp.float32),
    ],
)


def kernel(mean_divergence, max_divergence, per_unit_divergence,
           local_embeddings, unit_masks, attention_weights,
           W1, b1, W2, b2, W3, b3, U1, ub1, U2, ub2):
    embf = local_embeddings.reshape(B * N, D)
    vals16, idx16, div16, attn16, mask16, embk = _make_sc_topk()(
        per_unit_divergence, unit_masks, attention_weights, embf)

    emb_col = embk.reshape(B * KPAD, D)
    div_col = div16[:, :KPAD].reshape(B * KPAD, 1)
    vals_col = vals16[:, :KPAD].reshape(B * KPAD, 1)
    mask_col = mask16[:, :KPAD].reshape(B * KPAD, 1)

    u2d = (U2[:, 1] - U2[:, 0]).reshape(H, 1)
    ub2d = (ub2[1] - ub2[0]).reshape(1, 1)

    score, probs_col, flag_col, conf = _tc_mlp(
        mean_divergence.reshape(B, 1), max_divergence.reshape(B, 1),
        W1, b1.reshape(1, H), W2, b2.reshape(1, H // 2), W3, b3.reshape(1, 1),
        U1[:D], U1[D:], ub1.reshape(1, H), u2d, ub2d,
        emb_col, div_col, vals_col, mask_col)

    topk_vals = vals16[:, :K]
    topk_idx = idx16[:, :K]
    probs = probs_col.reshape(B, KPAD)[:, :K]
    flag = flag_col.reshape(B, KPAD)[:, :K] > 0.5
    attn_topk = attn16[:, :K]
    return (score[:, 0], topk_vals, topk_idx, probs, flag, conf[:, 0],
            attn_topk)


# speedup vs baseline: 1.1737x; 1.1737x over previous
import functools

import jax
import jax.numpy as jnp
from jax import lax
from jax.experimental import pallas as pl
from jax.experimental.pallas import tpu as pltpu
from jax.experimental.pallas import tpu_sc as plsc

B, N, D, H, K = 16, 2048, 768, 256, 5
THRESH = 0.5
LANES = 16
CHUNKS = N // LANES
KPAD = 8

_NC = 2
_NS = 16


def _sc_topk_body(pud_hbm, masks_hbm, attn_hbm, emb_hbm,
                  vals_out, idx_out, div_out, attn_out, mask_out, emb_out,
                  pud_v, masks_v, attn_v, masked_v, idx_buf, rows_v,
                  st_vals, st_idx, st_div, st_attn, st_mask, sem):
    c = lax.axis_index("c")
    s = lax.axis_index("s")
    wid = s * _NC + c

    @pl.when(wid < B)
    def _():
        b = wid
        pltpu.sync_copy(pud_hbm.at[b], pud_v)
        pltpu.sync_copy(masks_hbm.at[b], masks_v)
        pltpu.sync_copy(attn_hbm.at[b], attn_v)

        lane = lax.iota(jnp.int32, LANES)

        def _butterfly(mvec, ivec):
            for shift in (8, 4, 2, 1):
                st_vals[...] = mvec
                st_idx[...] = ivec
                perm = jnp.bitwise_xor(lane, shift)
                mv2 = plsc.load_gather(st_vals, [perm])
                iv2 = plsc.load_gather(st_idx, [perm])
                better = (mv2 > mvec) | ((mv2 == mvec) & (iv2 < ivec))
                mvec = jnp.where(better, mv2, mvec)
                ivec = jnp.where(better, iv2, ivec)
            return mvec, ivec

        def _pass1(i, carry):
            mvec, ivec = carry
            v = pud_v[pl.ds(i * LANES, LANES)]
            m = masks_v[pl.ds(i * LANES, LANES)]
            cv = jnp.where(m > 0.5, v, 0.0)
            masked_v[pl.ds(i * LANES, LANES)] = cv
            iv = lane + i * LANES
            gt = cv > mvec
            return jnp.where(gt, cv, mvec), jnp.where(gt, iv, ivec)

        mvec, ivec = lax.fori_loop(
            0, CHUNKS, _pass1,
            (jnp.full((LANES,), -1.0, jnp.float32),
             jnp.zeros((LANES,), jnp.int32)), unroll=8)
        mvec, ivec = _butterfly(mvec, ivec)
        vals_acc = jnp.where(lane == 0, mvec, 0.0)
        idx_acc = jnp.where(lane == 0, ivec, 0)
        plsc.store_scatter(masked_v, [ivec],
                           jnp.full((LANES,), -1.0, jnp.float32),
                           mask=lane == 0)

        for r in range(1, K):
            def _scan_chunk(i, carry):
                mvec, ivec = carry
                cv = masked_v[pl.ds(i * LANES, LANES)]
                iv = lane + i * LANES
                gt = cv > mvec
                return jnp.where(gt, cv, mvec), jnp.where(gt, iv, ivec)

            mvec, ivec = lax.fori_loop(
                0, CHUNKS, _scan_chunk,
                (jnp.full((LANES,), -1.0, jnp.float32),
                 jnp.zeros((LANES,), jnp.int32)), unroll=8)
            mvec, ivec = _butterfly(mvec, ivec)
            vals_acc = jnp.where(lane == r, mvec, vals_acc)
            idx_acc = jnp.where(lane == r, ivec, idx_acc)
            plsc.store_scatter(masked_v, [ivec],
                               jnp.full((LANES,), -1.0, jnp.float32),
                               mask=lane == 0)

        idx_safe = jnp.where(lane < K, idx_acc, 0)
        st_vals[...] = vals_acc
        st_idx[...] = idx_acc
        st_div[...] = plsc.load_gather(pud_v, [idx_safe])
        st_attn[...] = plsc.load_gather(attn_v, [idx_safe])
        st_mask[...] = plsc.load_gather(masks_v, [idx_safe])

        idx_buf[...] = idx_safe + b * N
        pltpu.async_copy(emb_hbm.at[idx_buf.at[pl.ds(0, KPAD)]], rows_v,
                         sem).wait()

        pltpu.sync_copy(st_vals, vals_out.at[b])
        pltpu.sync_copy(st_idx, idx_out.at[b])
        pltpu.sync_copy(st_div, div_out.at[b])
        pltpu.sync_copy(st_attn, attn_out.at[b])
        pltpu.sync_copy(st_mask, mask_out.at[b])
        pltpu.sync_copy(rows_v, emb_out.at[b])


@functools.cache
def _make_sc_topk():
  return pl.kernel(
    _sc_topk_body,
    out_type=[
        jax.ShapeDtypeStruct((B, LANES), jnp.float32),
        jax.ShapeDtypeStruct((B, LANES), jnp.int32),
        jax.ShapeDtypeStruct((B, LANES), jnp.float32),
        jax.ShapeDtypeStruct((B, LANES), jnp.float32),
        jax.ShapeDtypeStruct((B, LANES), jnp.float32),
        jax.ShapeDtypeStruct((B, KPAD, D), jnp.float32),
    ],
    mesh=plsc.VectorSubcoreMesh(core_axis_name="c", subcore_axis_name="s"),
    compiler_params=pltpu.CompilerParams(needs_layout_passes=False),
    scratch_types=[
        pltpu.VMEM((N,), jnp.float32),
        pltpu.VMEM((N,), jnp.float32),
        pltpu.VMEM((N,), jnp.float32),
        pltpu.VMEM((N,), jnp.float32),
        pltpu.VMEM((LANES,), jnp.int32),
        pltpu.VMEM((KPAD, D), jnp.float32),
        pltpu.VMEM((LANES,), jnp.float32),
        pltpu.VMEM((LANES,), jnp.int32),
        pltpu.VMEM((LANES,), jnp.float32),
        pltpu.VMEM((LANES,), jnp.float32),
        pltpu.VMEM((LANES,), jnp.float32),
        pltpu.SemaphoreType.DMA,
    ],
  )


def _tc_mlp_body(mean_ref, max_ref, w1_ref, b1_ref, w2_ref, b2_ref, w3_ref,
                 b3_ref, u1_ref, ub1_ref, u2_ref, ub2_ref,
                 vals_ref, idx_ref, div_ref, attn_ref, mask_ref, emb_ref,
                 score_out, vals_out, idx_out, probs_out, flag_out, conf_out,
                 attn_out):
    f32 = jnp.float32
    meanv = mean_ref[...]
    maxv = max_ref[...]
    w1 = w1_ref[...]
    const_row = 3.0 * w1[2:3, :] + 0.01 * w1[3:4, :] + b1_ref[...]
    h1 = jnp.maximum(meanv * w1[0:1, :] + maxv * w1[1:2, :] + const_row, 0.0)
    h2 = jnp.maximum(
        jnp.dot(h1, w2_ref[...], preferred_element_type=f32) + b2_ref[...],
        0.0)
    logit = jnp.dot(h2, w3_ref[...], preferred_element_type=f32) + b3_ref[...]
    score_out[...] = 1.0 / (1.0 + jnp.exp(-logit))

    u1a = u1_ref[pl.ds(0, D), :]
    u1row = u1_ref[pl.ds(D, 1), :]
    ub1 = ub1_ref[...]
    u2 = u2_ref[...]
    u2d = u2[:, 1:2] - u2[:, 0:1]
    ub2 = ub2_ref[...]
    ub2d = ub2[:, 1:2] - ub2[:, 0:1]
    divs = div_ref[...]
    pcols = []
    for j in range(K):
        embj = emb_ref[:, j, :]
        uh = jnp.maximum(
            jnp.dot(embj, u1a, preferred_element_type=f32)
            + divs[:, j:j + 1] * u1row + ub1, 0.0)
        ld = jnp.dot(uh, u2d, preferred_element_type=f32) + ub2d
        pcols.append(1.0 / (1.0 + jnp.exp(-ld)))
    probs = jnp.concatenate(pcols, axis=1)
    probs_out[...] = probs

    vals5 = vals_ref[...][:, :K]
    mask5 = mask_ref[...][:, :K]
    flagb = (mask5 > 0.5) & (vals5 > THRESH)
    flag_out[...] = flagb
    flagf = flagb.astype(f32)
    cnt = jnp.sum(flagf, axis=1, keepdims=True)
    sumpf = jnp.sum(probs * flagf, axis=1, keepdims=True)
    avg_conf = jnp.where(cnt > 0, sumpf / jnp.maximum(cnt, 1.0), 0.5)
    sep = (maxv - meanv) / (maxv + 1e-8)
    conf_out[...] = jnp.clip(0.4 * sep + 0.15 + 0.3 * avg_conf, 0.0, 1.0)

    vals_out[...] = vals5
    idx_out[...] = idx_ref[...][:, :K]
    attn_out[...] = attn_ref[...][:, :K]


_tc_mlp = pl.pallas_call(
    _tc_mlp_body,
    out_shape=[
        jax.ShapeDtypeStruct((B, 1), jnp.float32),
        jax.ShapeDtypeStruct((B, K), jnp.float32),
        jax.ShapeDtypeStruct((B, K), jnp.int32),
        jax.ShapeDtypeStruct((B, K), jnp.float32),
        jax.ShapeDtypeStruct((B, K), jnp.bool_),
        jax.ShapeDtypeStruct((B, 1), jnp.float32),
        jax.ShapeDtypeStruct((B, K), jnp.float32),
    ],
)


def kernel(mean_divergence, max_divergence, per_unit_divergence,
           local_embeddings, unit_masks, attention_weights,
           W1, b1, W2, b2, W3, b3, U1, ub1, U2, ub2):
    embf = local_embeddings.reshape(B * N, D)
    vals16, idx16, div16, attn16, mask16, embk = _make_sc_topk()(
        per_unit_divergence, unit_masks, attention_weights, embf)

    score, topk_vals, topk_idx, probs, flag, conf, attn_topk = _tc_mlp(
        mean_divergence.reshape(B, 1), max_divergence.reshape(B, 1),
        W1, b1.reshape(1, H), W2, b2.reshape(1, H // 2), W3, b3.reshape(1, 1),
        U1, ub1.reshape(1, H), U2, ub2.reshape(1, 2),
        vals16, idx16, div16, attn16, mask16, embk)

    return (score.reshape(B), topk_vals, topk_idx, probs, flag,
            conf.reshape(B), attn_topk)


# speedup vs baseline: 1.9860x; 1.6921x over previous
import jax
import jax.numpy as jnp
from jax import lax
from jax.experimental import pallas as pl
from jax.experimental.pallas import tpu as pltpu

B, N, D, H, K = 16, 2048, 768, 256, 5
THRESH = 0.5
KPAD = 8


def _fused_body(mean_ref, max_ref, w1_ref, b1_ref, w2_ref, b2_ref, w3_ref,
                b3_ref, u1_ref, ub1_ref, u2_ref, ub2_ref,
                pud_ref, masks_ref, attn_ref, emb_hbm,
                score_out, vals_out, idx_out, probs_out, flag_out, conf_out,
                attn_out,
                rows_v, idx_v, idx_s, sem_idx, sem_rows):
    f32 = jnp.float32
    i32 = jnp.int32

    pud = pud_ref[...]
    masks = masks_ref[...]
    attn = attn_ref[...]
    masked = jnp.where(masks > 0.5, pud, 0.0)
    iota2 = lax.broadcasted_iota(i32, (B, N), 1)

    vals_cols, idx_cols, div_cols, attn_cols, mask_cols = [], [], [], [], []
    for _ in range(K):
        rowmax = jnp.max(masked, axis=1, keepdims=True)
        cand = jnp.where(masked == rowmax, iota2, N)
        rowidx = jnp.min(cand, axis=1, keepdims=True)
        onehot = iota2 == rowidx
        div_cols.append(jnp.sum(jnp.where(onehot, pud, 0.0), axis=1,
                                keepdims=True))
        attn_cols.append(jnp.sum(jnp.where(onehot, attn, 0.0), axis=1,
                                 keepdims=True))
        mask_cols.append(jnp.sum(jnp.where(onehot, masks, 0.0), axis=1,
                                 keepdims=True))
        vals_cols.append(rowmax)
        idx_cols.append(rowidx)
        masked = jnp.where(onehot, -1.0, masked)

    vals5 = jnp.concatenate(vals_cols, axis=1)
    idx5 = jnp.concatenate(idx_cols, axis=1)
    div5 = jnp.concatenate(div_cols, axis=1)
    attn5 = jnp.concatenate(attn_cols, axis=1)
    mask5 = jnp.concatenate(mask_cols, axis=1)

    idx_v[...] = jnp.concatenate(
        idx_cols + [idx_cols[-1]] * (KPAD - K), axis=1)
    idx_copy = pltpu.make_async_copy(idx_v, idx_s, sem_idx)
    idx_copy.start()
    idx_copy.wait()

    copies = []
    for b in range(B):
        for j in range(K):
            copies.append(pltpu.make_async_copy(
                emb_hbm.at[b, idx_s[b, j]], rows_v.at[b, j], sem_rows))
    for c in copies:
        c.start()
    for c in copies:
        c.wait()

    meanv = mean_ref[...]
    maxv = max_ref[...]
    w1 = w1_ref[...]
    const_row = 3.0 * w1[2:3, :] + 0.01 * w1[3:4, :] + b1_ref[...]
    h1 = jnp.maximum(meanv * w1[0:1, :] + maxv * w1[1:2, :] + const_row, 0.0)
    h2 = jnp.maximum(
        jnp.dot(h1, w2_ref[...], preferred_element_type=f32) + b2_ref[...],
        0.0)
    logit = jnp.dot(h2, w3_ref[...], preferred_element_type=f32) + b3_ref[...]
    score_out[...] = 1.0 / (1.0 + jnp.exp(-logit))

    u1a = u1_ref[pl.ds(0, D), :]
    u1row = u1_ref[pl.ds(D, 1), :]
    ub1 = ub1_ref[...]
    u2 = u2_ref[...]
    u2d = u2[:, 1:2] - u2[:, 0:1]
    ub2 = ub2_ref[...]
    ub2d = ub2[:, 1:2] - ub2[:, 0:1]
    pcols = []
    for j in range(K):
        embj = rows_v[:, j, :]
        uh = jnp.maximum(
            jnp.dot(embj, u1a, preferred_element_type=f32)
            + div5[:, j:j + 1] * u1row + ub1, 0.0)
        ld = jnp.dot(uh, u2d, preferred_element_type=f32) + ub2d
        pcols.append(1.0 / (1.0 + jnp.exp(-ld)))
    probs = jnp.concatenate(pcols, axis=1)
    probs_out[...] = probs

    flagb = (mask5 > 0.5) & (vals5 > THRESH)
    flag_out[...] = flagb
    flagf = flagb.astype(f32)
    cnt = jnp.sum(flagf, axis=1, keepdims=True)
    sumpf = jnp.sum(probs * flagf, axis=1, keepdims=True)
    avg_conf = jnp.where(cnt > 0, sumpf / jnp.maximum(cnt, 1.0), 0.5)
    sep = (maxv - meanv) / (maxv + 1e-8)
    conf_out[...] = jnp.clip(0.4 * sep + 0.15 + 0.3 * avg_conf, 0.0, 1.0)

    vals_out[...] = vals5
    idx_out[...] = idx5
    attn_out[...] = attn5


_fused = pl.pallas_call(
    _fused_body,
    in_specs=[pl.BlockSpec(memory_space=pltpu.MemorySpace.HBM) if i == 15
              else pl.BlockSpec(memory_space=pltpu.MemorySpace.VMEM)
              for i in range(16)],
    out_shape=[
        jax.ShapeDtypeStruct((B, 1), jnp.float32),
        jax.ShapeDtypeStruct((B, K), jnp.float32),
        jax.ShapeDtypeStruct((B, K), jnp.int32),
        jax.ShapeDtypeStruct((B, K), jnp.float32),
        jax.ShapeDtypeStruct((B, K), jnp.bool_),
        jax.ShapeDtypeStruct((B, 1), jnp.float32),
        jax.ShapeDtypeStruct((B, K), jnp.float32),
    ],
    scratch_shapes=[
        pltpu.VMEM((B, KPAD, D), jnp.float32),
        pltpu.VMEM((B, KPAD), jnp.int32),
        pltpu.SMEM((B, KPAD), jnp.int32),
        pltpu.SemaphoreType.DMA,
        pltpu.SemaphoreType.DMA,
    ],
)


def kernel(mean_divergence, max_divergence, per_unit_divergence,
           local_embeddings, unit_masks, attention_weights,
           W1, b1, W2, b2, W3, b3, U1, ub1, U2, ub2):
    score, topk_vals, topk_idx, probs, flag, conf, attn_topk = _fused(
        mean_divergence.reshape(B, 1), max_divergence.reshape(B, 1),
        W1, b1.reshape(1, H), W2, b2.reshape(1, H // 2), W3, b3.reshape(1, 1),
        U1, ub1.reshape(1, H), U2, ub2.reshape(1, 2),
        per_unit_divergence, unit_masks, attention_weights, local_embeddings)

    return (score.reshape(B), topk_vals, topk_idx, probs, flag,
            conf.reshape(B), attn_topk)


# speedup vs baseline: 2.0896x; 1.0522x over previous
import jax
import jax.numpy as jnp
from jax import lax
from jax.experimental import pallas as pl
from jax.experimental.pallas import tpu as pltpu

B, N, D, H, K = 16, 2048, 768, 256, 5
THRESH = 0.5
KPAD = 8


def _fused_body(mean_ref, max_ref, w1_ref, b1_ref, w2_ref, b2_ref, w3_ref,
                b3_ref, u1_ref, ub1_ref, u2_ref, ub2_ref,
                pud_ref, masks_ref, attn_ref, emb_hbm,
                score_out, vals_out, idx_out, probs_out, flag_out, conf_out,
                attn_out,
                rows_v, idx_v, idx_s, sem_idx, sem_rows):
    f32 = jnp.float32
    i32 = jnp.int32

    pud = pud_ref[...]
    masks = masks_ref[...]
    attn = attn_ref[...]
    masked = jnp.where(masks > 0.5, pud, 0.0)
    iota2 = lax.broadcasted_iota(i32, (B, N), 1)

    vals_cols, idx_cols, div_cols, attn_cols = [], [], [], []
    for _ in range(K):
        rowmax = jnp.max(masked, axis=1, keepdims=True)
        cand = jnp.where(masked == rowmax, iota2, N)
        rowidx = jnp.min(cand, axis=1, keepdims=True)
        onehot = iota2 == rowidx
        div_cols.append(jnp.sum(jnp.where(onehot, pud, 0.0), axis=1,
                                keepdims=True))
        attn_cols.append(jnp.sum(jnp.where(onehot, attn, 0.0), axis=1,
                                 keepdims=True))
        vals_cols.append(rowmax)
        idx_cols.append(rowidx)
        masked = jnp.where(onehot, -1.0, masked)

    vals5 = jnp.concatenate(vals_cols, axis=1)
    idx5 = jnp.concatenate(idx_cols, axis=1)
    attn5 = jnp.concatenate(attn_cols, axis=1)

    idx_v[...] = jnp.concatenate(
        idx_cols + [idx_cols[-1]] * (KPAD - K), axis=1)
    idx_copy = pltpu.make_async_copy(idx_v, idx_s, sem_idx)
    idx_copy.start()
    idx_copy.wait()

    copies = []
    for j in range(K):
        for b in range(B):
            copies.append(pltpu.make_async_copy(
                emb_hbm.at[b, idx_s[b, j]], rows_v.at[j * B + b], sem_rows))
    for c in copies:
        c.start()

    meanv = mean_ref[...]
    maxv = max_ref[...]
    w1 = w1_ref[...]
    const_row = 3.0 * w1[2:3, :] + 0.01 * w1[3:4, :] + b1_ref[...]
    h1 = jnp.maximum(meanv * w1[0:1, :] + maxv * w1[1:2, :] + const_row, 0.0)
    h2 = jnp.maximum(
        jnp.dot(h1, w2_ref[...], preferred_element_type=f32) + b2_ref[...],
        0.0)
    logit = jnp.dot(h2, w3_ref[...], preferred_element_type=f32) + b3_ref[...]
    score_out[...] = 1.0 / (1.0 + jnp.exp(-logit))

    for c in copies:
        c.wait()

    u1a = u1_ref[pl.ds(0, D), :]
    u1row = u1_ref[pl.ds(D, 1), :]
    u2 = u2_ref[...]
    u2d = u2[:, 1:2] - u2[:, 0:1]
    ub2 = ub2_ref[...]
    ub2d = ub2[:, 1:2] - ub2[:, 0:1]
    divT = jnp.concatenate(div_cols, axis=0)
    emball = rows_v[...]
    uh = jnp.maximum(
        jnp.dot(emball, u1a, preferred_element_type=f32)
        + divT * u1row + ub1_ref[...], 0.0)
    ld = jnp.dot(uh, u2d, preferred_element_type=f32) + ub2d
    pcol = 1.0 / (1.0 + jnp.exp(-ld))
    probs = jnp.concatenate(
        [lax.slice(pcol, (j * B, 0), ((j + 1) * B, 1)) for j in range(K)],
        axis=1)
    probs_out[...] = probs

    flagb = vals5 > THRESH
    flag_out[...] = flagb
    flagf = flagb.astype(f32)
    cnt = jnp.sum(flagf, axis=1, keepdims=True)
    sumpf = jnp.sum(probs * flagf, axis=1, keepdims=True)
    avg_conf = jnp.where(cnt > 0, sumpf / jnp.maximum(cnt, 1.0), 0.5)
    sep = (maxv - meanv) / (maxv + 1e-8)
    conf_out[...] = jnp.clip(0.4 * sep + 0.15 + 0.3 * avg_conf, 0.0, 1.0)

    vals_out[...] = vals5
    idx_out[...] = idx5
    attn_out[...] = attn5


_fused = pl.pallas_call(
    _fused_body,
    in_specs=[pl.BlockSpec(memory_space=pltpu.MemorySpace.HBM) if i == 15
              else pl.BlockSpec(memory_space=pltpu.MemorySpace.VMEM)
              for i in range(16)],
    out_shape=[
        jax.ShapeDtypeStruct((B, 1), jnp.float32),
        jax.ShapeDtypeStruct((B, K), jnp.float32),
        jax.ShapeDtypeStruct((B, K), jnp.int32),
        jax.ShapeDtypeStruct((B, K), jnp.float32),
        jax.ShapeDtypeStruct((B, K), jnp.bool_),
        jax.ShapeDtypeStruct((B, 1), jnp.float32),
        jax.ShapeDtypeStruct((B, K), jnp.float32),
    ],
    scratch_shapes=[
        pltpu.VMEM((K * B, D), jnp.float32),
        pltpu.VMEM((B, KPAD), jnp.int32),
        pltpu.SMEM((B, KPAD), jnp.int32),
        pltpu.SemaphoreType.DMA,
        pltpu.SemaphoreType.DMA,
    ],
)


def kernel(mean_divergence, max_divergence, per_unit_divergence,
           local_embeddings, unit_masks, attention_weights,
           W1, b1, W2, b2, W3, b3, U1, ub1, U2, ub2):
    score, topk_vals, topk_idx, probs, flag, conf, attn_topk = _fused(
        mean_divergence.reshape(B, 1), max_divergence.reshape(B, 1),
        W1, b1.reshape(1, H), W2, b2.reshape(1, H // 2), W3, b3.reshape(1, 1),
        U1, ub1.reshape(1, H), U2, ub2.reshape(1, 2),
        per_unit_divergence, unit_masks, attention_weights, local_embeddings)

    return (score.reshape(B), topk_vals, topk_idx, probs, flag,
            conf.reshape(B), attn_topk)
